# baseline (speedup 1.0000x reference)
import jax
import jax.numpy as jnp
from jax.experimental import pallas as pl
from jax.experimental.pallas import tpu as pltpu

_CHUNKS = 5


def _embed_kernel(src_hbm, out_hbm, buf, in_sems, out_sems):
    rows = src_hbm.shape[0] // _CHUNKS
    for i in range(_CHUNKS):
        pltpu.make_async_copy(
            src_hbm.at[pl.ds(i * rows, rows), :],
            buf.at[pl.ds(i * rows, rows), :],
            in_sems.at[i],
        ).start()
    for i in range(_CHUNKS):
        pltpu.make_async_copy(
            src_hbm.at[pl.ds(i * rows, rows), :],
            buf.at[pl.ds(i * rows, rows), :],
            in_sems.at[i],
        ).wait()
        pltpu.make_async_copy(
            buf.at[pl.ds(i * rows, rows), :],
            out_hbm.at[0, pl.ds(i * rows, rows), :],
            out_sems.at[i],
        ).start()
    for i in range(_CHUNKS):
        pltpu.make_async_copy(
            buf.at[pl.ds(i * rows, rows), :],
            out_hbm.at[0, pl.ds(i * rows, rows), :],
            out_sems.at[i],
        ).wait()


def kernel(x, pos_table):
    maxlen = x.shape[-1]
    embed_dim = pos_table.shape[-1]
    return pl.pallas_call(
        _embed_kernel,
        in_specs=[pl.BlockSpec(memory_space=pl.ANY)],
        out_specs=pl.BlockSpec(memory_space=pl.ANY),
        out_shape=jax.ShapeDtypeStruct((1, maxlen, embed_dim), pos_table.dtype),
        scratch_shapes=[
            pltpu.VMEM((maxlen, embed_dim), pos_table.dtype),
            pltpu.SemaphoreType.DMA((_CHUNKS,)),
            pltpu.SemaphoreType.DMA((_CHUNKS,)),
        ],
    )(pos_table[:maxlen])


# speedup vs baseline: 1.0489x; 1.0489x over previous
import jax
import jax.numpy as jnp
from jax.experimental import pallas as pl
from jax.experimental.pallas import tpu as pltpu


def _embed_kernel(table_ref, out_hbm, sem):
    copy = pltpu.make_async_copy(table_ref, out_hbm.at[0], sem)
    copy.start()
    copy.wait()


def kernel(x, pos_table):
    maxlen = x.shape[-1]
    embed_dim = pos_table.shape[-1]
    return pl.pallas_call(
        _embed_kernel,
        in_specs=[pl.BlockSpec((maxlen, embed_dim), lambda: (0, 0))],
        out_specs=pl.BlockSpec(memory_space=pl.ANY),
        out_shape=jax.ShapeDtypeStruct((1, maxlen, embed_dim), pos_table.dtype),
        scratch_shapes=[pltpu.SemaphoreType.DMA],
    )(pos_table[:maxlen])


# speedup vs baseline: 1.0562x; 1.0069x over previous
import jax
import jax.numpy as jnp
from jax.experimental import pallas as pl
from jax.experimental.pallas import tpu as pltpu


def _embed_kernel(src_hbm, out_hbm, buf, sem):
    cin = pltpu.make_async_copy(src_hbm, buf, sem)
    cin.start()
    cin.wait()
    cout = pltpu.make_async_copy(buf, out_hbm.at[0], sem)
    cout.start()
    cout.wait()


def kernel(x, pos_table):
    maxlen = x.shape[-1]
    embed_dim = pos_table.shape[-1]
    return pl.pallas_call(
        _embed_kernel,
        in_specs=[pl.BlockSpec(memory_space=pl.ANY)],
        out_specs=pl.BlockSpec(memory_space=pl.ANY),
        out_shape=jax.ShapeDtypeStruct((1, maxlen, embed_dim), pos_table.dtype),
        scratch_shapes=[
            pltpu.VMEM((maxlen, embed_dim), pos_table.dtype),
            pltpu.SemaphoreType.DMA,
        ],
    )(pos_table[:maxlen])
